# Initial kernel scaffold; baseline (speedup 1.0000x reference)
import jax
import jax.numpy as jnp
from jax.experimental import pallas as pl


def kernel(inputs):
    raise NotImplementedError("write your pallas kernel here")


# speedup vs baseline: 1.7382x; 1.7382x over previous
import functools

import jax
import jax.numpy as jnp
from jax import lax
from jax.experimental import pallas as pl
from jax.experimental.pallas import tpu as pltpu
from jax.experimental.pallas import tpu_sc as plsc

DEPTH = 256
LANES = 16
NUM_CORES = 2
NUM_SUBCORES = 16
NUM_WORKERS = NUM_CORES * NUM_SUBCORES


def _build_scatter_kernel(n_idx: int):
    assert n_idx % NUM_WORKERS == 0
    idx_per_worker = n_idx // NUM_WORKERS
    idx_per_chunk = 192
    vec_per_chunk = idx_per_chunk // LANES
    floats_per_chunk = idx_per_chunk * DEPTH
    assert idx_per_worker % idx_per_chunk == 0
    n_chunks = idx_per_worker // idx_per_chunk
    assert n_chunks % 2 == 0
    out_len = n_idx * DEPTH

    mesh = plsc.VectorSubcoreMesh(
        core_axis_name="c",
        subcore_axis_name="s",
        num_cores=NUM_CORES,
        num_subcores=NUM_SUBCORES,
    )

    @functools.partial(
        pl.kernel,
        out_type=jax.ShapeDtypeStruct((out_len,), jnp.float32),
        mesh=mesh,
        compiler_params=pltpu.CompilerParams(needs_layout_passes=False),
        scratch_types=[
            pltpu.VMEM((floats_per_chunk,), jnp.float32),
            pltpu.VMEM((floats_per_chunk,), jnp.float32),
            pltpu.VMEM((idx_per_worker,), jnp.int32),
            pltpu.SemaphoreType.DMA,
            pltpu.SemaphoreType.DMA,
        ],
    )
    def scatter_kernel(in_hbm, out_hbm, buf0, buf1, idx_all, sem0, sem1):
        wid = lax.axis_index("s") * NUM_CORES + lax.axis_index("c")
        in_base = wid * idx_per_worker
        out_base = wid * idx_per_worker * DEPTH

        bufs = (buf0, buf1)
        sems = (sem0, sem1)
        lane_off = lax.iota(jnp.int32, 16) * DEPTH
        ones = jnp.full((LANES,), 1.0, jnp.float32)
        zeros_v = jnp.zeros((LANES,), jnp.float32)

        pltpu.sync_copy(in_hbm.at[pl.ds(in_base, idx_per_worker)], idx_all)

        @pl.loop(0, floats_per_chunk // (LANES * 8))
        def _(i):
            for u in range(8):
                off = (i * 8 + u) * LANES
                buf0[pl.ds(off, LANES)] = zeros_v
                buf1[pl.ds(off, LANES)] = zeros_v

        def scatter(buf, c, val):
            for jv in range(vec_per_chunk):
                iv = idx_all[pl.ds(c * idx_per_chunk + jv * LANES, LANES)]
                pv = lane_off + (jv * LANES * DEPTH) + iv
                plsc.store_scatter(buf, [pv], val)

        def emit(c, b):
            scatter(bufs[b], c, ones)
            pltpu.async_copy(
                bufs[b],
                out_hbm.at[pl.ds(out_base + c * floats_per_chunk,
                                 floats_per_chunk)],
                sems[b],
            )

        def drain(b):
            pltpu.make_async_copy(
                bufs[b],
                out_hbm.at[pl.ds(out_base, floats_per_chunk)],
                sems[b],
            ).wait()

        emit(0, 0)
        emit(1, 1)

        @pl.loop(1, n_chunks // 2)
        def _(g):
            for b in range(2):
                drain(b)
                scatter(bufs[b], 2 * (g - 1) + b, zeros_v)
                emit(2 * g + b, b)

        drain(0)
        drain(1)

    return scatter_kernel


def kernel(inputs):
    B, W, H, D = inputs.shape
    n_idx = B * W * H * D
    flat = inputs.astype(jnp.int32).reshape(n_idx)
    out = _build_scatter_kernel(n_idx)(flat)
    return out.reshape(B, W, H, DEPTH * D)
